# baseline (speedup 1.0000x reference)
import jax
import jax.numpy as jnp
from jax import lax
from jax.experimental import pallas as pl
from jax.experimental.pallas import tpu as pltpu
from jax.experimental.pallas import tpu_sc as plsc

N_FEATURES = 26
BATCH = 16384
EMBED_DIM = 32
VOCAB = 100000
OUT_DIM = N_FEATURES * EMBED_DIM

_NC, _NS = 2, 16
_NW = _NC * _NS
_CB = 128
_NFULL = VOCAB // _CB
_TAIL = VOCAB - _NFULL * _CB
_TROWS = VOCAB * EMBED_DIM


def _bodyA(*refs):
    tabs = refs[:N_FEATURES]
    tails = refs[N_FEATURES:2 * N_FEATURES]
    out = refs[2 * N_FEATURES]
    blkA, blkB, rowA, rowB = refs[2 * N_FEATURES + 1:2 * N_FEATURES + 5]
    ginA, ginB, woutA, woutB = refs[2 * N_FEATURES + 5:2 * N_FEATURES + 9]

    wid = lax.axis_index("s") * _NC + lax.axis_index("c")
    lane32 = jax.lax.broadcasted_iota(jnp.int32, (16,), 0) * 32
    nmine = (_NFULL - wid + _NW - 1) // _NW
    npairs = nmine * (N_FEATURES // 2)

    def fire_in(q, blk, gsem):
        i = q // N_FEATURES
        f = q % N_FEATURES
        c0 = (wid + i * _NW) * _CB
        for F in range(N_FEATURES):
            @pl.when(f == F)
            def _():
                pltpu.async_copy(tabs[F].at[:, pl.ds(c0, _CB)], blk, gsem)

    def drain_blk(blk, sem):
        pltpu.make_async_copy(tabs[0].at[:, pl.ds(0, _CB)], blk, sem).wait()

    def drain_row(row, sem):
        pltpu.make_async_copy(out.at[pl.ds(0, _CB * 32)], row, sem).wait()

    def transpose(blk, row):
        def kbody(k, _):
            base = k * 512
            for j in range(32):
                vals = blk[j, pl.ds(k * 16, 16)]
                plsc.store_scatter(row, [lane32 + (base + j)], vals)
            return 0
        lax.fori_loop(0, _CB // 16, kbody, 0)

    def fire_out(q, row, wsem):
        i = q // N_FEATURES
        f = q % N_FEATURES
        b = wid + i * _NW
        pltpu.async_copy(
            row, out.at[pl.ds(f * _TROWS + b * (_CB * 32), _CB * 32)], wsem)

    def pair(p, _):
        q0 = 2 * p
        q1 = q0 + 1
        fire_in(q0, blkA, ginA)
        fire_in(q1, blkB, ginB)

        @pl.when(p >= 1)
        def _():
            drain_row(rowA, woutA)
        drain_blk(blkA, ginA)
        transpose(blkA, rowA)
        fire_out(q0, rowA, woutA)

        @pl.when(p >= 1)
        def _():
            drain_row(rowB, woutB)
        drain_blk(blkB, ginB)
        transpose(blkB, rowB)
        fire_out(q1, rowB, woutB)
        return 0

    lax.fori_loop(0, npairs, pair, 0)

    @pl.when(npairs >= 1)
    def _():
        drain_row(rowA, woutA)
        drain_row(rowB, woutB)

    @pl.when(wid == 0)
    def _():
        for f in range(N_FEATURES):
            pltpu.sync_copy(
                tails[f],
                out.at[pl.ds(f * _TROWS + _NFULL * _CB * 32, _TAIL * 32)])


_mesh = plsc.VectorSubcoreMesh(core_axis_name="c", subcore_axis_name="s")

_callA = pl.kernel(
    _bodyA,
    out_type=jax.ShapeDtypeStruct((N_FEATURES * _TROWS,), jnp.float32),
    mesh=_mesh,
    scratch_types=(
        [pltpu.VMEM((32, _CB), jnp.float32)] * 2
        + [pltpu.VMEM((_CB * 32,), jnp.float32)] * 2
        + [pltpu.SemaphoreType.DMA] * 4
    ),
    compiler_params=pltpu.CompilerParams(
        use_tc_tiling_on_sc=True, needs_layout_passes=False),
)

_B_W = BATCH // _NW
_CHUNK = 128
_N_CHUNK = _B_W // _CHUNK


def _bodyB(*refs):
    idx_refs = refs[:N_FEATURES]
    tab = refs[N_FEATURES]
    out = refs[N_FEATURES + 1]
    idx_v = refs[N_FEATURES + 2]
    bufs = refs[N_FEATURES + 3]
    gsem = refs[N_FEATURES + 4]

    wid = lax.axis_index("s") * _NC + lax.axis_index("c")
    base = wid * _B_W

    def start_feature(f, slot):
        pltpu.sync_copy(idx_refs[f].at[pl.ds(base, _B_W)], idx_v.at[slot])
        if f:
            off = f * VOCAB
            for k in range(_B_W // 16):
                sl = pl.ds(k * 16, 16)
                idx_v[slot, sl] = idx_v[slot, sl] + off
        descs = []
        for c in range(_N_CHUNK):
            descs.append(pltpu.async_copy(
                tab.at[idx_v.at[slot, pl.ds(c * _CHUNK, _CHUNK)]],
                bufs.at[slot, pl.ds(c * _CHUNK, _CHUNK)],
                gsem))
        return descs

    def finish_feature(f, slot, descs):
        for d in descs:
            d.wait()
        pltpu.sync_copy(
            bufs.at[slot],
            out.at[pl.ds(base, _B_W), pl.ds(f * EMBED_DIM, EMBED_DIM)])

    descs = start_feature(0, 0)
    for f in range(1, N_FEATURES):
        nxt = start_feature(f, f % 2)
        finish_feature(f - 1, (f - 1) % 2, descs)
        descs = nxt
    finish_feature(N_FEATURES - 1, (N_FEATURES - 1) % 2, descs)


_callB = pl.kernel(
    _bodyB,
    out_type=jax.ShapeDtypeStruct((BATCH, OUT_DIM), jnp.float32),
    mesh=_mesh,
    scratch_types=[
        pltpu.VMEM((2, _B_W), jnp.int32),
        pltpu.VMEM((2, _B_W, EMBED_DIM), jnp.float32),
        pltpu.SemaphoreType.DMA,
    ],
    compiler_params=pltpu.CompilerParams(use_tc_tiling_on_sc=False),
)


def kernel(*args):
    idxs = args[:N_FEATURES]
    tables = args[N_FEATURES:2 * N_FEATURES]
    tts = [jnp.transpose(t) for t in tables]
    tails = [t[_NFULL * _CB:, :].reshape(-1) for t in tables]
    flat = _callA(*tts, *tails)
    stack = flat.reshape(N_FEATURES * VOCAB, EMBED_DIM)
    return _callB(*idxs, stack)


# speedup vs baseline: 1.0018x; 1.0018x over previous
import jax
import jax.numpy as jnp
from jax import lax
from jax.experimental import pallas as pl
from jax.experimental.pallas import tpu as pltpu
from jax.experimental.pallas import tpu_sc as plsc

N_FEATURES = 26
BATCH = 16384
EMBED_DIM = 32
VOCAB = 100000
OUT_DIM = N_FEATURES * EMBED_DIM

_NC, _NS = 2, 16
_NW = _NC * _NS
_CB = 128
_NFULL = VOCAB // _CB
_TAIL = VOCAB - _NFULL * _CB
_TROWS = VOCAB * EMBED_DIM


def _bodyA(*refs):
    tabs = refs[:N_FEATURES]
    tails = refs[N_FEATURES:2 * N_FEATURES]
    out = refs[2 * N_FEATURES]
    blkA, blkB, rowA, rowB = refs[2 * N_FEATURES + 1:2 * N_FEATURES + 5]
    ginA, ginB, woutA, woutB = refs[2 * N_FEATURES + 5:2 * N_FEATURES + 9]

    wid = lax.axis_index("s") * _NC + lax.axis_index("c")
    lane32 = jax.lax.broadcasted_iota(jnp.int32, (16,), 0) * 32
    jvecs = [lane32 + j for j in range(32)]
    nmine = (_NFULL - wid + _NW - 1) // _NW
    npairs = nmine * (N_FEATURES // 2)

    def fire_in(q, blk, gsem):
        i = q // N_FEATURES
        f = q % N_FEATURES
        c0 = (wid + i * _NW) * _CB
        for F in range(N_FEATURES):
            @pl.when(f == F)
            def _():
                pltpu.async_copy(tabs[F].at[:, pl.ds(c0, _CB)], blk, gsem)

    def drain_blk(blk, sem):
        pltpu.make_async_copy(tabs[0].at[:, pl.ds(0, _CB)], blk, sem).wait()

    def drain_row(row, sem):
        pltpu.make_async_copy(out.at[pl.ds(0, _CB * 32)], row, sem).wait()

    def transpose(blk, row):
        for k in range(_CB // 16):
            dst = row.at[pl.ds(k * 512, 512)]
            for j in range(32):
                plsc.store_scatter(dst, [jvecs[j]], blk[j, pl.ds(k * 16, 16)])

    def fire_out(q, row, wsem):
        i = q // N_FEATURES
        f = q % N_FEATURES
        b = wid + i * _NW
        pltpu.async_copy(
            row, out.at[pl.ds(f * _TROWS + b * (_CB * 32), _CB * 32)], wsem)

    def pair(p, _):
        q0 = 2 * p
        q1 = q0 + 1
        fire_in(q0, blkA, ginA)
        fire_in(q1, blkB, ginB)

        @pl.when(p >= 1)
        def _():
            drain_row(rowA, woutA)
        drain_blk(blkA, ginA)
        transpose(blkA, rowA)
        fire_out(q0, rowA, woutA)

        @pl.when(p >= 1)
        def _():
            drain_row(rowB, woutB)
        drain_blk(blkB, ginB)
        transpose(blkB, rowB)
        fire_out(q1, rowB, woutB)
        return 0

    lax.fori_loop(0, npairs, pair, 0)

    @pl.when(npairs >= 1)
    def _():
        drain_row(rowA, woutA)
        drain_row(rowB, woutB)

    @pl.when(wid == 0)
    def _():
        for f in range(N_FEATURES):
            pltpu.sync_copy(
                tails[f],
                out.at[pl.ds(f * _TROWS + _NFULL * _CB * 32, _TAIL * 32)])


_mesh = plsc.VectorSubcoreMesh(core_axis_name="c", subcore_axis_name="s")

_callA = pl.kernel(
    _bodyA,
    out_type=jax.ShapeDtypeStruct((N_FEATURES * _TROWS,), jnp.float32),
    mesh=_mesh,
    scratch_types=(
        [pltpu.VMEM((32, _CB), jnp.float32)] * 2
        + [pltpu.VMEM((_CB * 32,), jnp.float32)] * 2
        + [pltpu.SemaphoreType.DMA] * 4
    ),
    compiler_params=pltpu.CompilerParams(
        use_tc_tiling_on_sc=True, needs_layout_passes=False),
)

_B_W = BATCH // _NW
_CHUNK = 128
_N_CHUNK = _B_W // _CHUNK


def _bodyB(*refs):
    idx_refs = refs[:N_FEATURES]
    tab = refs[N_FEATURES]
    out = refs[N_FEATURES + 1]
    idx_v = refs[N_FEATURES + 2]
    bufs = refs[N_FEATURES + 3]
    gsem = refs[N_FEATURES + 4]

    wid = lax.axis_index("s") * _NC + lax.axis_index("c")
    base = wid * _B_W

    def start_feature(f, slot):
        pltpu.sync_copy(idx_refs[f].at[pl.ds(base, _B_W)], idx_v.at[slot])
        if f:
            off = f * VOCAB
            for k in range(_B_W // 16):
                sl = pl.ds(k * 16, 16)
                idx_v[slot, sl] = idx_v[slot, sl] + off
        descs = []
        for c in range(_N_CHUNK):
            descs.append(pltpu.async_copy(
                tab.at[idx_v.at[slot, pl.ds(c * _CHUNK, _CHUNK)]],
                bufs.at[slot, pl.ds(c * _CHUNK, _CHUNK)],
                gsem))
        return descs

    def finish_feature(f, slot, descs):
        for d in descs:
            d.wait()
        pltpu.sync_copy(
            bufs.at[slot],
            out.at[pl.ds(base, _B_W), pl.ds(f * EMBED_DIM, EMBED_DIM)])

    descs = start_feature(0, 0)
    for f in range(1, N_FEATURES):
        nxt = start_feature(f, f % 2)
        finish_feature(f - 1, (f - 1) % 2, descs)
        descs = nxt
    finish_feature(N_FEATURES - 1, (N_FEATURES - 1) % 2, descs)


_callB = pl.kernel(
    _bodyB,
    out_type=jax.ShapeDtypeStruct((BATCH, OUT_DIM), jnp.float32),
    mesh=_mesh,
    scratch_types=[
        pltpu.VMEM((2, _B_W), jnp.int32),
        pltpu.VMEM((2, _B_W, EMBED_DIM), jnp.float32),
        pltpu.SemaphoreType.DMA,
    ],
    compiler_params=pltpu.CompilerParams(use_tc_tiling_on_sc=False),
)


def kernel(*args):
    idxs = args[:N_FEATURES]
    tables = args[N_FEATURES:2 * N_FEATURES]
    tts = [jnp.transpose(t) for t in tables]
    tails = [t[_NFULL * _CB:, :].reshape(-1) for t in tables]
    flat = _callA(*tts, *tails)
    stack = flat.reshape(N_FEATURES * VOCAB, EMBED_DIM)
    return _callB(*idxs, stack)


# speedup vs baseline: 1.1505x; 1.1485x over previous
import jax
import jax.numpy as jnp
from jax import lax
from jax.experimental import pallas as pl
from jax.experimental.pallas import tpu as pltpu
from jax.experimental.pallas import tpu_sc as plsc

N_FEATURES = 26
BATCH = 16384
EMBED_DIM = 32
VOCAB = 100000
OUT_DIM = N_FEATURES * EMBED_DIM

_NC, _NS = 2, 16
_NW = _NC * _NS
_CB = 128
_NFULL = VOCAB // _CB
_TAIL = VOCAB - _NFULL * _CB
_TROWS = VOCAB * EMBED_DIM


def _bodyA(*refs):
    tabs = refs[:N_FEATURES]
    tails = refs[N_FEATURES:2 * N_FEATURES]
    out = refs[2 * N_FEATURES]
    blks = refs[2 * N_FEATURES + 1:2 * N_FEATURES + 5]
    rows = refs[2 * N_FEATURES + 5:2 * N_FEATURES + 9]
    gins = refs[2 * N_FEATURES + 9:2 * N_FEATURES + 13]
    wouts = refs[2 * N_FEATURES + 13:2 * N_FEATURES + 17]

    wid = lax.axis_index("s") * _NC + lax.axis_index("c")
    lane32 = jax.lax.broadcasted_iota(jnp.int32, (16,), 0) * 32
    jvecs = [lane32 + j for j in range(32)]
    nmine = (_NFULL - wid + _NW - 1) // _NW
    nitems = nmine * N_FEATURES
    nquads = (nitems + 3) // 4

    def fire_in(q, s):
        i = q // N_FEATURES
        f = q % N_FEATURES
        c0 = (wid + i * _NW) * _CB
        for F in range(N_FEATURES):
            @pl.when(f == F)
            def _():
                pltpu.async_copy(tabs[F].at[:, pl.ds(c0, _CB)], blks[s],
                                 gins[s])

    def drain_blk(s):
        pltpu.make_async_copy(tabs[0].at[:, pl.ds(0, _CB)], blks[s],
                              gins[s]).wait()

    def drain_row(s):
        pltpu.make_async_copy(out.at[pl.ds(0, _CB * 32)], rows[s],
                              wouts[s]).wait()

    def transpose(s):
        def kbody(k, _):
            dst = rows[s].at[pl.ds(k * 512, 512)]
            for j in range(32):
                plsc.store_scatter(dst, [jvecs[j]],
                                   blks[s][j, pl.ds(k * 16, 16)])
            return 0
        lax.fori_loop(0, _CB // 16, kbody, 0)

    def fire_out(q, s):
        i = q // N_FEATURES
        f = q % N_FEATURES
        b = wid + i * _NW
        pltpu.async_copy(
            rows[s],
            out.at[pl.ds(f * _TROWS + b * (_CB * 32), _CB * 32)], wouts[s])

    for s in range(4):
        @pl.when(s < nitems)
        def _(s=s):
            fire_in(s, s)

    def quad(p, _):
        for s in range(4):
            q = 4 * p + s

            @pl.when((p > 0) & (q - 4 < nitems))
            def _(s=s):
                drain_row(s)

            @pl.when(q < nitems)
            def _(s=s, q=q):
                drain_blk(s)
            transpose(s)

            @pl.when(q < nitems)
            def _(s=s, q=q):
                fire_out(q, s)

            @pl.when(q + 4 < nitems)
            def _(s=s, q=q):
                fire_in(q + 4, s)
        return 0

    lax.fori_loop(0, nquads, quad, 0)

    for s in range(4):
        @pl.when(4 * (nquads - 1) + s < nitems)
        def _(s=s):
            drain_row(s)

    for f in range(N_FEATURES):
        @pl.when(wid == f % _NW)
        def _(f=f):
            pltpu.sync_copy(
                tails[f],
                out.at[pl.ds(f * _TROWS + _NFULL * _CB * 32, _TAIL * 32)])


_mesh = plsc.VectorSubcoreMesh(core_axis_name="c", subcore_axis_name="s")

_callA = pl.kernel(
    _bodyA,
    out_type=jax.ShapeDtypeStruct((N_FEATURES * _TROWS,), jnp.float32),
    mesh=_mesh,
    scratch_types=(
        [pltpu.VMEM((32, _CB), jnp.float32)] * 4
        + [pltpu.VMEM((_CB * 32,), jnp.float32)] * 4
        + [pltpu.SemaphoreType.DMA] * 8
    ),
    compiler_params=pltpu.CompilerParams(
        use_tc_tiling_on_sc=True, needs_layout_passes=False),
)

_B_W = BATCH // _NW
_CHUNK = 128
_N_CHUNK = _B_W // _CHUNK


def _bodyB(*refs):
    idx_refs = refs[:N_FEATURES]
    tab = refs[N_FEATURES]
    out = refs[N_FEATURES + 1]
    idx_v = refs[N_FEATURES + 2]
    bufs = refs[N_FEATURES + 3]
    gsem = refs[N_FEATURES + 4]

    wid = lax.axis_index("s") * _NC + lax.axis_index("c")
    base = wid * _B_W

    def start_feature(f, slot):
        pltpu.sync_copy(idx_refs[f].at[pl.ds(base, _B_W)], idx_v.at[slot])
        if f:
            off = f * VOCAB
            for k in range(_B_W // 16):
                sl = pl.ds(k * 16, 16)
                idx_v[slot, sl] = idx_v[slot, sl] + off
        descs = []
        for c in range(_N_CHUNK):
            descs.append(pltpu.async_copy(
                tab.at[idx_v.at[slot, pl.ds(c * _CHUNK, _CHUNK)]],
                bufs.at[slot, pl.ds(c * _CHUNK, _CHUNK)],
                gsem))
        return descs

    def finish_feature(f, slot, descs):
        for d in descs:
            d.wait()
        pltpu.sync_copy(
            bufs.at[slot],
            out.at[pl.ds(base, _B_W), pl.ds(f * EMBED_DIM, EMBED_DIM)])

    descs = start_feature(0, 0)
    for f in range(1, N_FEATURES):
        nxt = start_feature(f, f % 2)
        finish_feature(f - 1, (f - 1) % 2, descs)
        descs = nxt
    finish_feature(N_FEATURES - 1, (N_FEATURES - 1) % 2, descs)


_callB = pl.kernel(
    _bodyB,
    out_type=jax.ShapeDtypeStruct((BATCH, OUT_DIM), jnp.float32),
    mesh=_mesh,
    scratch_types=[
        pltpu.VMEM((2, _B_W), jnp.int32),
        pltpu.VMEM((2, _B_W, EMBED_DIM), jnp.float32),
        pltpu.SemaphoreType.DMA,
    ],
    compiler_params=pltpu.CompilerParams(use_tc_tiling_on_sc=False),
)


def kernel(*args):
    idxs = args[:N_FEATURES]
    tables = args[N_FEATURES:2 * N_FEATURES]
    tts = [jnp.transpose(t) for t in tables]
    tails = [t[_NFULL * _CB:, :].reshape(-1) for t in tables]
    flat = _callA(*tts, *tails)
    stack = flat.reshape(N_FEATURES * VOCAB, EMBED_DIM)
    return _callB(*idxs, stack)


# speedup vs baseline: 1.7494x; 1.5205x over previous
import jax
import jax.numpy as jnp
from jax import lax
from jax.experimental import pallas as pl
from jax.experimental.pallas import tpu as pltpu
from jax.experimental.pallas import tpu_sc as plsc

N_FEATURES = 26
BATCH = 16384
EMBED_DIM = 32
OUT_DIM = N_FEATURES * EMBED_DIM

_NC, _NS = 2, 16
_NW = _NC * _NS
_B_W = BATCH // _NW
_CHUNK = 128
_N_CHUNK = _B_W // _CHUNK


def _body(*refs):
    idx_refs = refs[:N_FEATURES]
    tab_refs = refs[N_FEATURES:2 * N_FEATURES]
    out = refs[2 * N_FEATURES]
    idx_v = refs[2 * N_FEATURES + 1]
    bufs = refs[2 * N_FEATURES + 2]
    gsem = refs[2 * N_FEATURES + 3]

    wid = lax.axis_index("s") * _NC + lax.axis_index("c")
    base = wid * _B_W

    def start_feature(f, slot):
        pltpu.sync_copy(idx_refs[f].at[pl.ds(base, _B_W)], idx_v.at[slot])
        descs = []
        for c in range(_N_CHUNK):
            descs.append(pltpu.async_copy(
                tab_refs[f].at[idx_v.at[slot, pl.ds(c * _CHUNK, _CHUNK)]],
                bufs.at[slot, pl.ds(c * _CHUNK, _CHUNK)],
                gsem))
        return descs

    def finish_feature(f, slot, descs):
        for d in descs:
            d.wait()
        pltpu.sync_copy(
            bufs.at[slot],
            out.at[pl.ds(base, _B_W), pl.ds(f * EMBED_DIM, EMBED_DIM)])

    descs = start_feature(0, 0)
    for f in range(1, N_FEATURES):
        next_descs = start_feature(f, f % 2)
        finish_feature(f - 1, (f - 1) % 2, descs)
        descs = next_descs
    finish_feature(N_FEATURES - 1, (N_FEATURES - 1) % 2, descs)


_mesh = plsc.VectorSubcoreMesh(core_axis_name="c", subcore_axis_name="s")

_sc_call = pl.kernel(
    _body,
    out_type=jax.ShapeDtypeStruct((BATCH, OUT_DIM), jnp.float32),
    mesh=_mesh,
    scratch_types=[
        pltpu.VMEM((2, _B_W), jnp.int32),
        pltpu.VMEM((2, _B_W, EMBED_DIM), jnp.float32),
        pltpu.SemaphoreType.DMA,
    ],
    compiler_params=pltpu.CompilerParams(use_tc_tiling_on_sc=False),
)


def kernel(*args):
    return _sc_call(*args)
